# baseline (speedup 1.0000x reference)
import functools

import jax
import jax.numpy as jnp
from jax import lax
from jax.experimental import pallas as pl
from jax.experimental.pallas import tpu as pltpu
from jax.experimental.pallas import tpu_sc as plsc

BATCH = 16384
HIST = 200
N = BATCH * HIST

_INFO = plsc.get_sparse_core_info()
NC = _INFO.num_cores
NS = _INFO.num_subcores
L = _INFO.num_lanes
NW = NC * NS

PER_W = N // NW
CHUNK = 5120
NCHUNK = PER_W // CHUNK
SEG = 6400


def _make_lookup(vocab: int):
    mesh = plsc.VectorSubcoreMesh(core_axis_name="c", subcore_axis_name="s")

    @functools.partial(
        pl.kernel,
        mesh=mesh,
        out_type=jax.ShapeDtypeStruct((N,), jnp.int32),
        scratch_types=[
            pltpu.VMEM_SHARED((vocab,), jnp.int32),
            pltpu.VMEM((vocab,), jnp.int32),
            pltpu.VMEM((CHUNK,), jnp.int32),
            pltpu.VMEM((CHUNK,), jnp.int32),
            pltpu.VMEM((CHUNK,), jnp.int32),
            pltpu.VMEM((CHUNK,), jnp.int32),
            pltpu.SemaphoreType.DMA,
            pltpu.SemaphoreType.DMA,
            pltpu.SemaphoreType.DMA,
            pltpu.SemaphoreType.DMA,
        ],
        compiler_params=pltpu.CompilerParams(
            needs_layout_passes=False,
            use_tc_tiling_on_sc=False,
        ),
    )
    def lookup(
        idx_hbm, tab_hbm, out_hbm,
        tab_sp, tab_v, in_a, in_b, res_a, res_b, s_ia, s_ib, s_oa, s_ob,
    ):
        sid = lax.axis_index("s")
        wid = sid * NC + lax.axis_index("c")
        base = wid * PER_W

        pltpu.async_copy(idx_hbm.at[pl.ds(base, CHUNK)], in_a, s_ia)
        pltpu.async_copy(idx_hbm.at[pl.ds(base + CHUNK, CHUNK)], in_b, s_ib)

        tail = vocab - (NS - 1) * SEG

        @pl.when(sid < NS - 1)
        def _():
            pltpu.sync_copy(
                tab_hbm.at[pl.ds(sid * SEG, SEG)],
                tab_sp.at[pl.ds(sid * SEG, SEG)],
            )

        @pl.when(sid == NS - 1)
        def _():
            pltpu.sync_copy(
                tab_hbm.at[pl.ds((NS - 1) * SEG, tail)],
                tab_sp.at[pl.ds((NS - 1) * SEG, tail)],
            )

        plsc.subcore_barrier()

        pltpu.sync_copy(tab_sp, tab_v)
        lane = lax.iota(jnp.int32, 16)
        head = tab_v[pl.ds(0, L)]
        tab_v[pl.ds(0, L)] = jnp.where(lane == 0, jnp.zeros_like(head), head)

        def gather_chunk(in_v, res_v):
            @plsc.parallel_loop(0, CHUNK, step=L, unroll=4)
            def _(o):
                res_v[pl.ds(o, L)] = plsc.load_gather(
                    tab_v, [in_v[pl.ds(o, L)]]
                )

        bufs = ((in_a, res_a, s_ia, s_oa), (in_b, res_b, s_ib, s_ob))

        def pair_body(g, carry):
            for p in range(2):
                in_v, res_v, s_in, s_out = bufs[p]
                k = 2 * g + p
                off = base + k * CHUNK
                pltpu.make_async_copy(
                    idx_hbm.at[pl.ds(off, CHUNK)], in_v, s_in
                ).wait()

                @pl.when(k >= 2)
                def _():
                    pltpu.make_async_copy(
                        res_v, out_hbm.at[pl.ds(off - 2 * CHUNK, CHUNK)], s_out
                    ).wait()

                gather_chunk(in_v, res_v)
                pltpu.async_copy(res_v, out_hbm.at[pl.ds(off, CHUNK)], s_out)

                @pl.when(k + 2 < NCHUNK)
                def _():
                    pltpu.async_copy(
                        idx_hbm.at[pl.ds(off + 2 * CHUNK, CHUNK)], in_v, s_in
                    )

            return carry

        lax.fori_loop(0, NCHUNK // 2, pair_body, 0)
        for k in (NCHUNK - 2, NCHUNK - 1):
            in_v, res_v, s_in, s_out = bufs[k % 2]
            pltpu.make_async_copy(
                res_v, out_hbm.at[pl.ds(base + k * CHUNK, CHUNK)], s_out
            ).wait()

    return lookup


def kernel(inputs, table_vals):
    idx = inputs.astype(jnp.int32)
    idx = idx.reshape(128, 128, 25, 8).transpose(2, 0, 3, 1).reshape(N)
    out = _make_lookup(table_vals.shape[0])(idx, table_vals)
    out = out.reshape(25, 128, 8, 128).transpose(1, 3, 0, 2)
    return out.reshape(BATCH, HIST)


# speedup vs baseline: 1.0071x; 1.0071x over previous
import functools

import jax
import jax.numpy as jnp
from jax import lax
from jax.experimental import pallas as pl
from jax.experimental.pallas import tpu as pltpu
from jax.experimental.pallas import tpu_sc as plsc

BATCH = 16384
HIST = 200
N = BATCH * HIST

_INFO = plsc.get_sparse_core_info()
NC = _INFO.num_cores
NS = _INFO.num_subcores
L = _INFO.num_lanes
NW = NC * NS

PER_W = N // NW
CHUNK = 5120
NCHUNK = PER_W // CHUNK
SEG = 6400


def _make_lookup(vocab: int):
    mesh = plsc.VectorSubcoreMesh(core_axis_name="c", subcore_axis_name="s")

    @functools.partial(
        pl.kernel,
        mesh=mesh,
        out_type=jax.ShapeDtypeStruct((N,), jnp.int32),
        scratch_types=[
            pltpu.VMEM_SHARED((vocab,), jnp.int32),
            pltpu.VMEM((vocab,), jnp.int32),
            pltpu.VMEM((CHUNK,), jnp.int32),
            pltpu.VMEM((CHUNK,), jnp.int32),
            pltpu.VMEM((CHUNK,), jnp.int32),
            pltpu.VMEM((CHUNK,), jnp.int32),
            pltpu.SemaphoreType.DMA,
            pltpu.SemaphoreType.DMA,
            pltpu.SemaphoreType.DMA,
            pltpu.SemaphoreType.DMA,
        ],
        compiler_params=pltpu.CompilerParams(
            needs_layout_passes=False,
            use_tc_tiling_on_sc=False,
        ),
    )
    def lookup(
        idx_hbm, tab_hbm, out_hbm,
        tab_sp, tab_v, in_a, in_b, res_a, res_b, s_ia, s_ib, s_oa, s_ob,
    ):
        sid = lax.axis_index("s")
        wid = sid * NC + lax.axis_index("c")
        base = wid * PER_W

        pltpu.async_copy(idx_hbm.at[pl.ds(base, CHUNK)], in_a, s_ia)
        pltpu.async_copy(idx_hbm.at[pl.ds(base + CHUNK, CHUNK)], in_b, s_ib)

        tail = vocab - (NS - 1) * SEG

        @pl.when(sid < NS - 1)
        def _():
            pltpu.sync_copy(
                tab_hbm.at[pl.ds(sid * SEG, SEG)],
                tab_sp.at[pl.ds(sid * SEG, SEG)],
            )

        @pl.when(sid == NS - 1)
        def _():
            pltpu.sync_copy(
                tab_hbm.at[pl.ds((NS - 1) * SEG, tail)],
                tab_sp.at[pl.ds((NS - 1) * SEG, tail)],
            )

        plsc.subcore_barrier()

        pltpu.sync_copy(tab_sp, tab_v)
        lane = lax.iota(jnp.int32, 16)
        head = tab_v[pl.ds(0, L)]
        tab_v[pl.ds(0, L)] = jnp.where(lane == 0, jnp.zeros_like(head), head)

        def gather_chunk(in_v, res_v):
            @plsc.parallel_loop(0, CHUNK, step=L, unroll=8)
            def _(o):
                res_v[pl.ds(o, L)] = plsc.load_gather(
                    tab_v, [in_v[pl.ds(o, L)]]
                )

        bufs = ((in_a, res_a, s_ia, s_oa), (in_b, res_b, s_ib, s_ob))

        def pair_body(g, carry):
            for p in range(2):
                in_v, res_v, s_in, s_out = bufs[p]
                k = 2 * g + p
                off = base + k * CHUNK
                pltpu.make_async_copy(
                    idx_hbm.at[pl.ds(off, CHUNK)], in_v, s_in
                ).wait()

                @pl.when(k >= 2)
                def _():
                    pltpu.make_async_copy(
                        res_v, out_hbm.at[pl.ds(off - 2 * CHUNK, CHUNK)], s_out
                    ).wait()

                gather_chunk(in_v, res_v)
                pltpu.async_copy(res_v, out_hbm.at[pl.ds(off, CHUNK)], s_out)

                @pl.when(k + 2 < NCHUNK)
                def _():
                    pltpu.async_copy(
                        idx_hbm.at[pl.ds(off + 2 * CHUNK, CHUNK)], in_v, s_in
                    )

            return carry

        lax.fori_loop(0, NCHUNK // 2, pair_body, 0)
        for k in (NCHUNK - 2, NCHUNK - 1):
            in_v, res_v, s_in, s_out = bufs[k % 2]
            pltpu.make_async_copy(
                res_v, out_hbm.at[pl.ds(base + k * CHUNK, CHUNK)], s_out
            ).wait()

    return lookup


def kernel(inputs, table_vals):
    idx = inputs.astype(jnp.int32)
    idx = idx.reshape(128, 128, 25, 8).transpose(2, 0, 3, 1).reshape(N)
    out = _make_lookup(table_vals.shape[0])(idx, table_vals)
    out = out.reshape(25, 128, 8, 128).transpose(1, 3, 0, 2)
    return out.reshape(BATCH, HIST)
